# baseline (speedup 1.0000x reference)
import jax
import jax.numpy as jnp
from jax import lax
from jax.experimental import pallas as pl
from jax.experimental.pallas import tpu as pltpu
from jax.experimental.pallas import tpu_sc as plsc

N = 50000
D = 256
S = 512
NLANE = 16
NT = D // NLANE
NW = 32
SEG_PER_W = S // NW
CHUNK = 120


def _body(x_hbm, starts_hbm, wb_hbm, out_hbm, starts_v, wb_v, xb0, xb1, xb2,
          gbuf, acc_v, sem0, sem1, sem2):
    c = lax.axis_index("c")
    s = lax.axis_index("s")
    w = s * 2 + c
    base = SEG_PER_W * w

    pltpu.sync_copy(starts_hbm, starts_v)
    pltpu.sync_copy(wb_hbm, wb_v)

    lanes = lax.iota(jnp.int32, NLANE)
    va = starts_v[pl.ds(base, NLANE)]
    vb = starts_v[pl.ds(base + 1, NLANE)]
    r0_all = va[0]
    r1_all = starts_v[pl.ds(base + SEG_PER_W, NLANE)][0]
    bvec = wb_v[pl.ds(D, NLANE)]
    wv = [wb_v[pl.ds(NLANE * t, NLANE)] for t in range(NT)]

    zero16 = jnp.zeros((NLANE,), jnp.float32)
    ninf16 = jnp.full((NLANE,), -jnp.inf, jnp.float32)
    for k in range(SEG_PER_W):
        for t in range(NT):
            acc_v[k, pl.ds(NLANE * t, NLANE)] = zero16
            acc_v[k, pl.ds(D + NLANE * t, NLANE)] = ninf16

    c0 = (r0_all // 8) * 8
    nchunks = (r1_all - c0 + CHUNK - 1) // CHUNK
    nch3 = (nchunks + 2) // 3

    bufs = (xb0, xb1, xb2)
    sems = (sem0, sem1, sem2)

    def _src(ci):
        sj = jnp.minimum(c0 + ci * CHUNK, N - CHUNK)
        sj = pl.multiple_of(sj, 8)
        return x_hbm.at[pl.ds(sj, CHUNK)]

    def _start(ci, u):
        pltpu.async_copy(_src(ci), bufs[u].at[pl.ds(0, CHUNK)], sems[u])

    def _wait(ci, u):
        pltpu.make_async_copy(_src(ci), bufs[u].at[pl.ds(0, CHUNK)],
                              sems[u]).wait()

    def _process(ci, xb):
        s_i = jnp.minimum(c0 + ci * CHUNK, N - CHUNK)
        lo = jnp.maximum(r0_all, c0 + ci * CHUNK)
        hi = jnp.minimum(r1_all, c0 + (ci + 1) * CHUNK)
        hi = jnp.maximum(hi, lo)

        def _gate(xs):
            pp = [xs[t] * wv[t] for t in range(NT)]
            while len(pp) > 1:
                pp = [pp[i2] + pp[i2 + 1] for i2 in range(0, len(pp), 2)]
            zv = pp[0]
            for m in (8, 4, 2, 1):
                zv = zv + zv.at[lanes ^ m].get(
                    mode="promise_in_bounds", unique_indices=True)
            return 1.0 / (1.0 + jnp.exp(-(zv + bvec)))

        def _bsum_i32(v):
            for m in (8, 4, 2, 1):
                v = v + v.at[lanes ^ m].get(
                    mode="promise_in_bounds", unique_indices=True)
            return v

        ks = _bsum_i32(jnp.where(vb <= lo, 1, 0))[0]
        ke = _bsum_i32(jnp.where(va < hi, 1, 0))[0]

        @pl.loop(ks, ke)
        def _seg(k):
            b0 = starts_v[pl.ds(base + k, NLANE)][0]
            b1 = starts_v[pl.ds(base + k + 1, NLANE)][0]
            a = jnp.maximum(b0, lo)
            e = jnp.minimum(b1, hi)
            e = jnp.maximum(e, a)
            sacc = tuple(acc_v[k, pl.ds(NLANE * t, NLANE)]
                         for t in range(NT))
            macc = tuple(acc_v[k, pl.ds(D + NLANE * t, NLANE)]
                         for t in range(NT))

            ngb = (e - a + 1) // 2

            @pl.loop(0, ngb, init_carry=(sacc, macc))
            def _row_b(i2, carry):
                sa, ma = carry
                jj = a + 2 * i2 - s_i
                x0 = [xb[jj, pl.ds(NLANE * t, NLANE)] for t in range(NT)]
                g0 = _gate(x0)
                valid = (a + 2 * i2 + 1) < e
                x1 = [xb[jj + 1, pl.ds(NLANE * t, NLANE)]
                      for t in range(NT)]
                g1 = jnp.where(valid, _gate(x1), zero16)
                x1m = [jnp.where(valid, x1[t], ninf16) for t in range(NT)]
                x1z = [jnp.where(valid, x1[t], zero16) for t in range(NT)]
                sa = tuple(sa[t] + g0 * x0[t] + g1 * x1z[t]
                           for t in range(NT))
                ma = tuple(jnp.maximum(jnp.maximum(ma[t], x0[t]), x1m[t])
                           for t in range(NT))
                return (sa, ma)

            sacc, macc = _row_b
            for t in range(NT):
                acc_v[k, pl.ds(NLANE * t, NLANE)] = sacc[t]
                acc_v[k, pl.ds(D + NLANE * t, NLANE)] = macc[t]

    for u in range(3):
        _start(u, u)

    @pl.loop(0, nch3)
    def _chunk3(i3):
        for u in range(3):
            ci = 3 * i3 + u
            _wait(ci, u)
            _process(ci, bufs[u])
            _start(ci + 3, u)

    for u in range(3):
        _wait(3 * nch3 + u, u)

    pltpu.sync_copy(acc_v, out_hbm.at[pl.ds(SEG_PER_W * w, SEG_PER_W)])


_mesh = plsc.VectorSubcoreMesh(core_axis_name="c", subcore_axis_name="s")

_sc_call = pl.kernel(
    _body,
    out_type=jax.ShapeDtypeStruct((S, 2 * D), jnp.float32),
    mesh=_mesh,
    scratch_types=[
        pltpu.VMEM((544,), jnp.int32),
        pltpu.VMEM((272,), jnp.float32),
        pltpu.VMEM((CHUNK + 4, D), jnp.float32),
        pltpu.VMEM((CHUNK + 4, D), jnp.float32),
        pltpu.VMEM((CHUNK + 4, D), jnp.float32),
        pltpu.VMEM((CHUNK + 4, NLANE), jnp.float32),
        pltpu.VMEM((SEG_PER_W, 2 * D), jnp.float32),
        pltpu.SemaphoreType.DMA,
        pltpu.SemaphoreType.DMA,
        pltpu.SemaphoreType.DMA,
    ],
)


def kernel(x, batch, W, b):
    batch32 = batch.astype(jnp.int32)
    ids = jnp.arange(S + 1, dtype=jnp.int32)
    starts = jnp.searchsorted(batch32, ids).astype(jnp.int32)
    starts = jnp.concatenate([starts, jnp.zeros((31,), jnp.int32)])
    wb = jnp.concatenate([
        W.reshape(-1).astype(jnp.float32),
        jnp.broadcast_to(b.astype(jnp.float32), (16,)),
    ])
    return _sc_call(x, starts, wb)


# speedup vs baseline: 1.4334x; 1.4334x over previous
import jax
import jax.numpy as jnp
from jax import lax
from jax.experimental import pallas as pl
from jax.experimental.pallas import tpu as pltpu
from jax.experimental.pallas import tpu_sc as plsc

N = 50000
D = 256
S = 512
NLANE = 16
NT = D // NLANE
NW = 32
SEG_PER_W = S // NW
CHUNK = 120
BR = 2000
NB = N // BR


def _tc_body(x_ref, b3_ref, w_ref, bias_ref, out_ref):
    i = pl.program_id(0)

    @pl.when(i == 0)
    def _():
        out_ref[...] = jnp.zeros_like(out_ref)

    xb = x_ref[...]
    bb = b3_ref[0, 0, :]
    logits = lax.dot_general(xb, w_ref[...], (((1,), (1,)), ((), ())),
                             preferred_element_type=jnp.float32)
    g = jax.nn.sigmoid(logits[:, 0:1] + bias_ref[0, 0])
    wx = g * xb
    oh = (bb[:, None] ==
          lax.broadcasted_iota(jnp.int32, (BR, S), 1)).astype(jnp.float32)
    out_ref[...] += lax.dot_general(oh, wx, (((0,), (0,)), ((), ())),
                                    preferred_element_type=jnp.float32)


_tc_call = pl.pallas_call(
    _tc_body,
    grid=(NB,),
    in_specs=[
        pl.BlockSpec((BR, D), lambda i: (i, 0)),
        pl.BlockSpec((1, 1, BR), lambda i: (i, 0, 0)),
        pl.BlockSpec((128, D), lambda i: (0, 0)),
        pl.BlockSpec((1, 1), lambda i: (0, 0)),
    ],
    out_specs=pl.BlockSpec((S, D), lambda i: (0, 0)),
    out_shape=jax.ShapeDtypeStruct((S, D), jnp.float32),
)


def _body(x_hbm, starts_hbm, out_hbm, starts_v, xb0, xb1, acc_v, sem0, sem1):
    c = lax.axis_index("c")
    s = lax.axis_index("s")
    w = c * 16 + s
    base = SEG_PER_W * w

    pltpu.sync_copy(starts_hbm, starts_v)

    lanes = lax.iota(jnp.int32, NLANE)
    va = starts_v[pl.ds(base, NLANE)]
    vb = starts_v[pl.ds(base + 1, NLANE)]
    r0_all = va[0]
    r1_all = starts_v[pl.ds(base + SEG_PER_W, NLANE)][0]

    ninf16 = jnp.full((NLANE,), -jnp.inf, jnp.float32)
    for k in range(SEG_PER_W):
        for t in range(NT):
            acc_v[k, pl.ds(NLANE * t, NLANE)] = ninf16

    c0 = (r0_all // 8) * 8
    nchunks = (r1_all - c0 + CHUNK - 1) // CHUNK
    nch2 = (nchunks + 1) // 2

    bufs = (xb0, xb1)
    sems = (sem0, sem1)

    def _src(ci):
        sj = jnp.minimum(c0 + ci * CHUNK, N - CHUNK)
        sj = pl.multiple_of(sj, 8)
        return x_hbm.at[pl.ds(sj, CHUNK)]

    def _start(ci, u):
        pltpu.async_copy(_src(ci), bufs[u].at[pl.ds(0, CHUNK)], sems[u])

    def _wait(ci, u):
        pltpu.make_async_copy(_src(ci), bufs[u].at[pl.ds(0, CHUNK)],
                              sems[u]).wait()

    def _process(ci, xb):
        s_i = jnp.minimum(c0 + ci * CHUNK, N - CHUNK)
        lo = jnp.maximum(r0_all, c0 + ci * CHUNK)
        hi = jnp.minimum(r1_all, c0 + (ci + 1) * CHUNK)
        hi = jnp.maximum(hi, lo)

        def _bsum_i32(v):
            for m in (8, 4, 2, 1):
                v = v + v.at[lanes ^ m].get(
                    mode="promise_in_bounds", unique_indices=True)
            return v

        ks = _bsum_i32(jnp.where(vb <= lo, 1, 0))[0]
        ke = _bsum_i32(jnp.where(va < hi, 1, 0))[0]

        @pl.loop(ks, ke)
        def _seg(k):
            b0 = starts_v[pl.ds(base + k, NLANE)][0]
            b1 = starts_v[pl.ds(base + k + 1, NLANE)][0]
            a = jnp.maximum(b0, lo)
            e = jnp.minimum(b1, hi)
            e = jnp.maximum(e, a)
            macc = tuple(acc_v[k, pl.ds(NLANE * t, NLANE)]
                         for t in range(NT))

            ngb = (e - a + 1) // 2

            @pl.loop(0, ngb, init_carry=macc)
            def _row_b(i2, ma):
                jj = a + 2 * i2 - s_i
                x0 = [xb[jj, pl.ds(NLANE * t, NLANE)] for t in range(NT)]
                valid = (a + 2 * i2 + 1) < e
                x1 = [xb[jj + 1, pl.ds(NLANE * t, NLANE)]
                      for t in range(NT)]
                x1m = [jnp.where(valid, x1[t], ninf16) for t in range(NT)]
                return tuple(
                    jnp.maximum(jnp.maximum(ma[t], x0[t]), x1m[t])
                    for t in range(NT))

            macc = _row_b
            for t in range(NT):
                acc_v[k, pl.ds(NLANE * t, NLANE)] = macc[t]

    for u in range(2):
        _start(u, u)

    @pl.loop(0, nch2)
    def _chunk2(i2):
        for u in range(2):
            ci = 2 * i2 + u
            _wait(ci, u)
            _process(ci, bufs[u])
            _start(ci + 2, u)

    for u in range(2):
        _wait(2 * nch2 + u, u)

    pltpu.sync_copy(acc_v, out_hbm.at[pl.ds(SEG_PER_W * w, SEG_PER_W)])


_mesh = plsc.VectorSubcoreMesh(core_axis_name="c", subcore_axis_name="s")

_sc_call = pl.kernel(
    _body,
    out_type=jax.ShapeDtypeStruct((S, D), jnp.float32),
    mesh=_mesh,
    scratch_types=[
        pltpu.VMEM((544,), jnp.int32),
        pltpu.VMEM((CHUNK + 4, D), jnp.float32),
        pltpu.VMEM((CHUNK + 4, D), jnp.float32),
        pltpu.VMEM((SEG_PER_W, D), jnp.float32),
        pltpu.SemaphoreType.DMA,
        pltpu.SemaphoreType.DMA,
    ],
)


def kernel(x, batch, W, b):
    batch32 = batch.astype(jnp.int32)
    ids = jnp.arange(S + 1, dtype=jnp.int32)
    starts = jnp.searchsorted(batch32, ids).astype(jnp.int32)
    starts = jnp.concatenate([starts, jnp.zeros((31,), jnp.int32)])
    wpad = jnp.zeros((128, D), jnp.float32).at[0].set(
        W.reshape(-1).astype(jnp.float32))
    out1 = _tc_call(x, batch32.reshape(NB, 1, BR), wpad,
                    b.astype(jnp.float32).reshape(1, 1))
    out2 = _sc_call(x, starts)
    return jnp.concatenate([out1, out2], axis=1)


# speedup vs baseline: 1.4335x; 1.0001x over previous
import jax
import jax.numpy as jnp
from jax import lax
from jax.experimental import pallas as pl
from jax.experimental.pallas import tpu as pltpu
from jax.experimental.pallas import tpu_sc as plsc

N = 50000
D = 256
S = 512
NLANE = 16
NT = D // NLANE
NW = 32
SEG_PER_W = S // NW
CHUNK = 120
BR = 2000
NB = N // BR


def _tc_body(x_ref, b3_ref, w_ref, bias_ref, out_ref):
    i = pl.program_id(0)

    @pl.when(i == 0)
    def _():
        out_ref[...] = jnp.zeros_like(out_ref)

    xb = x_ref[...]
    bb = b3_ref[0, 0, :]
    logits = lax.dot_general(xb, w_ref[...], (((1,), (1,)), ((), ())),
                             preferred_element_type=jnp.float32)
    g = jax.nn.sigmoid(logits[:, 0:1] + bias_ref[0, 0])
    wx = g * xb
    oh = (bb[:, None] ==
          lax.broadcasted_iota(jnp.int32, (BR, S), 1)).astype(jnp.float32)
    out_ref[...] += lax.dot_general(oh, wx, (((0,), (0,)), ((), ())),
                                    preferred_element_type=jnp.float32)


_tc_call = pl.pallas_call(
    _tc_body,
    grid=(NB,),
    in_specs=[
        pl.BlockSpec((BR, D), lambda i: (i, 0)),
        pl.BlockSpec((1, 1, BR), lambda i: (i, 0, 0)),
        pl.BlockSpec((128, D), lambda i: (0, 0)),
        pl.BlockSpec((1, 1), lambda i: (0, 0)),
    ],
    out_specs=pl.BlockSpec((S, D), lambda i: (0, 0)),
    out_shape=jax.ShapeDtypeStruct((S, D), jnp.float32),
)


def _body(x_hbm, starts_hbm, out_hbm, starts_v, xb0, xb1, acc_v, sem0, sem1):
    c = lax.axis_index("c")
    s = lax.axis_index("s")
    w = c * 16 + s
    base = SEG_PER_W * w

    pltpu.sync_copy(starts_hbm, starts_v)

    lanes = lax.iota(jnp.int32, NLANE)
    va = starts_v[pl.ds(base, NLANE)]
    vb = starts_v[pl.ds(base + 1, NLANE)]
    r0_all = va[0]
    r1_all = starts_v[pl.ds(base + SEG_PER_W, NLANE)][0]

    ninf16 = jnp.full((NLANE,), -jnp.inf, jnp.float32)
    for k in range(SEG_PER_W):
        for t in range(NT):
            acc_v[k, pl.ds(NLANE * t, NLANE)] = ninf16

    c0 = (r0_all // 8) * 8
    nchunks = (r1_all - c0 + CHUNK - 1) // CHUNK
    nch2 = (nchunks + 1) // 2

    bufs = (xb0, xb1)
    sems = (sem0, sem1)

    def _src(ci):
        sj = jnp.minimum(c0 + ci * CHUNK, N - CHUNK)
        sj = pl.multiple_of(sj, 8)
        return x_hbm.at[pl.ds(sj, CHUNK)]

    def _start(ci, u):
        pltpu.async_copy(_src(ci), bufs[u].at[pl.ds(0, CHUNK)], sems[u])

    def _wait(ci, u):
        pltpu.make_async_copy(_src(ci), bufs[u].at[pl.ds(0, CHUNK)],
                              sems[u]).wait()

    def _process(ci, xb):
        s_i = jnp.minimum(c0 + ci * CHUNK, N - CHUNK)
        lo = jnp.maximum(r0_all, c0 + ci * CHUNK)
        hi = jnp.minimum(r1_all, c0 + (ci + 1) * CHUNK)
        hi = jnp.maximum(hi, lo)

        def _bsum_i32(v):
            for m in (8, 4, 2, 1):
                v = v + v.at[lanes ^ m].get(
                    mode="promise_in_bounds", unique_indices=True)
            return v

        ks = _bsum_i32(jnp.where(vb <= lo, 1, 0))[0]
        ke = _bsum_i32(jnp.where(va < hi, 1, 0))[0]

        @pl.loop(ks, ke)
        def _seg(k):
            b0 = starts_v[pl.ds(base + k, NLANE)][0]
            b1 = starts_v[pl.ds(base + k + 1, NLANE)][0]
            a = jnp.maximum(b0, lo)
            e = jnp.minimum(b1, hi)
            e = jnp.maximum(e, a)
            macc = tuple(acc_v[k, pl.ds(NLANE * t, NLANE)]
                         for t in range(NT))

            ngb = (e - a + 1) // 2

            @pl.loop(0, ngb, init_carry=macc)
            def _row_b(i2, ma):
                jj = a + 2 * i2 - s_i
                x0 = [xb[jj, pl.ds(NLANE * t, NLANE)] for t in range(NT)]
                valid = (a + 2 * i2 + 1) < e
                x1 = [xb[jj + 1, pl.ds(NLANE * t, NLANE)]
                      for t in range(NT)]
                x1m = [jnp.where(valid, x1[t], ninf16) for t in range(NT)]
                return tuple(
                    jnp.maximum(jnp.maximum(ma[t], x0[t]), x1m[t])
                    for t in range(NT))

            macc = _row_b
            for t in range(NT):
                acc_v[k, pl.ds(NLANE * t, NLANE)] = macc[t]

    for u in range(2):
        _start(u, u)

    @pl.loop(0, nch2)
    def _chunk2(i2):
        for u in range(2):
            ci = 2 * i2 + u
            _wait(ci, u)
            _process(ci, bufs[u])
            _start(ci + 2, u)

    for u in range(2):
        _wait(2 * nch2 + u, u)

    pltpu.sync_copy(acc_v, out_hbm.at[pl.ds(SEG_PER_W * w, SEG_PER_W)])


_mesh = plsc.VectorSubcoreMesh(core_axis_name="c", subcore_axis_name="s")

_sc_call = pl.kernel(
    _body,
    out_type=jax.ShapeDtypeStruct((S, D), jnp.float32),
    mesh=_mesh,
    scratch_types=[
        pltpu.VMEM((544,), jnp.int32),
        pltpu.VMEM((CHUNK + 4, D), jnp.float32),
        pltpu.VMEM((CHUNK + 4, D), jnp.float32),
        pltpu.VMEM((SEG_PER_W, D), jnp.float32),
        pltpu.SemaphoreType.DMA,
        pltpu.SemaphoreType.DMA,
    ],
)


def kernel(x, batch, W, b):
    batch32 = batch.astype(jnp.int32)
    ids = jnp.arange(S + 1, dtype=jnp.int32)
    starts = jnp.searchsorted(batch32, ids).astype(jnp.int32)
    starts = jnp.concatenate([starts, jnp.zeros((31,), jnp.int32)])
    wpad = jnp.zeros((128, D), jnp.float32).at[0].set(
        W.reshape(-1).astype(jnp.float32))
    out2 = _sc_call(x, starts)
    out1 = _tc_call(x, batch32.reshape(NB, 1, BR), wpad,
                    b.astype(jnp.float32).reshape(1, 1))
    return jnp.concatenate([out1, out2], axis=1)


# speedup vs baseline: 1.4499x; 1.0114x over previous
import jax
import jax.numpy as jnp
from jax import lax
from jax.experimental import pallas as pl
from jax.experimental.pallas import tpu as pltpu
from jax.experimental.pallas import tpu_sc as plsc

N = 50000
D = 256
S = 512
NLANE = 16
NT = D // NLANE
NW = 32
SEG_PER_W = S // NW
CHUNK = 120
BR = 2000
NB = N // BR


def _tc_body(x_ref, b3_ref, w_ref, bias_ref, out_ref):
    i = pl.program_id(0)

    @pl.when(i == 0)
    def _():
        out_ref[...] = jnp.zeros_like(out_ref)

    xb = x_ref[...]
    bb = b3_ref[0, 0, :]
    logits = lax.dot_general(xb, w_ref[...], (((1,), (1,)), ((), ())),
                             preferred_element_type=jnp.float32)
    g = jax.nn.sigmoid(logits[:, 0:1] + bias_ref[0, 0])
    wx = (g * xb).astype(jnp.bfloat16)
    oh = (bb[:, None] ==
          lax.broadcasted_iota(jnp.int32, (BR, S), 1)).astype(jnp.bfloat16)
    out_ref[...] += lax.dot_general(oh, wx, (((0,), (0,)), ((), ())),
                                    preferred_element_type=jnp.float32)


_tc_call = pl.pallas_call(
    _tc_body,
    grid=(NB,),
    in_specs=[
        pl.BlockSpec((BR, D), lambda i: (i, 0)),
        pl.BlockSpec((1, 1, BR), lambda i: (i, 0, 0)),
        pl.BlockSpec((128, D), lambda i: (0, 0)),
        pl.BlockSpec((1, 1), lambda i: (0, 0)),
    ],
    out_specs=pl.BlockSpec((S, D), lambda i: (0, 0)),
    out_shape=jax.ShapeDtypeStruct((S, D), jnp.float32),
)


def _body(x_hbm, starts_hbm, out_hbm, starts_v, xb0, xb1, acc_v, sem0, sem1):
    c = lax.axis_index("c")
    s = lax.axis_index("s")
    w = c * 16 + s
    base = SEG_PER_W * w

    pltpu.sync_copy(starts_hbm, starts_v)

    lanes = lax.iota(jnp.int32, NLANE)
    va = starts_v[pl.ds(base, NLANE)]
    vb = starts_v[pl.ds(base + 1, NLANE)]
    r0_all = va[0]
    r1_all = starts_v[pl.ds(base + SEG_PER_W, NLANE)][0]

    ninf16 = jnp.full((NLANE,), -jnp.inf, jnp.float32)
    for k in range(SEG_PER_W):
        for t in range(NT):
            acc_v[k, pl.ds(NLANE * t, NLANE)] = ninf16

    c0 = (r0_all // 8) * 8
    nchunks = (r1_all - c0 + CHUNK - 1) // CHUNK
    nch2 = (nchunks + 1) // 2

    bufs = (xb0, xb1)
    sems = (sem0, sem1)

    def _src(ci):
        sj = jnp.minimum(c0 + ci * CHUNK, N - CHUNK)
        sj = pl.multiple_of(sj, 8)
        return x_hbm.at[pl.ds(sj, CHUNK)]

    def _start(ci, u):
        pltpu.async_copy(_src(ci), bufs[u].at[pl.ds(0, CHUNK)], sems[u])

    def _wait(ci, u):
        pltpu.make_async_copy(_src(ci), bufs[u].at[pl.ds(0, CHUNK)],
                              sems[u]).wait()

    def _process(ci, xb):
        s_i = jnp.minimum(c0 + ci * CHUNK, N - CHUNK)
        lo = jnp.maximum(r0_all, c0 + ci * CHUNK)
        hi = jnp.minimum(r1_all, c0 + (ci + 1) * CHUNK)
        hi = jnp.maximum(hi, lo)

        def _bsum_i32(v):
            for m in (8, 4, 2, 1):
                v = v + v.at[lanes ^ m].get(
                    mode="promise_in_bounds", unique_indices=True)
            return v

        ks = _bsum_i32(jnp.where(vb <= lo, 1, 0))[0]
        ke = _bsum_i32(jnp.where(va < hi, 1, 0))[0]

        @pl.loop(ks, ke)
        def _seg(k):
            b0 = starts_v[pl.ds(base + k, NLANE)][0]
            b1 = starts_v[pl.ds(base + k + 1, NLANE)][0]
            a = jnp.maximum(b0, lo)
            e = jnp.minimum(b1, hi)
            e = jnp.maximum(e, a)
            macc = tuple(acc_v[k, pl.ds(NLANE * t, NLANE)]
                         for t in range(NT))

            ngb = (e - a + 1) // 2

            @pl.loop(0, ngb, init_carry=macc)
            def _row_b(i2, ma):
                jj = a + 2 * i2 - s_i
                x0 = [xb[jj, pl.ds(NLANE * t, NLANE)] for t in range(NT)]
                valid = (a + 2 * i2 + 1) < e
                x1 = [xb[jj + 1, pl.ds(NLANE * t, NLANE)]
                      for t in range(NT)]
                x1m = [jnp.where(valid, x1[t], ninf16) for t in range(NT)]
                return tuple(
                    jnp.maximum(jnp.maximum(ma[t], x0[t]), x1m[t])
                    for t in range(NT))

            macc = _row_b
            for t in range(NT):
                acc_v[k, pl.ds(NLANE * t, NLANE)] = macc[t]

    for u in range(2):
        _start(u, u)

    @pl.loop(0, nch2)
    def _chunk2(i2):
        for u in range(2):
            ci = 2 * i2 + u
            _wait(ci, u)
            _process(ci, bufs[u])
            _start(ci + 2, u)

    for u in range(2):
        _wait(2 * nch2 + u, u)

    pltpu.sync_copy(acc_v, out_hbm.at[pl.ds(SEG_PER_W * w, SEG_PER_W)])


_mesh = plsc.VectorSubcoreMesh(core_axis_name="c", subcore_axis_name="s")

_sc_call = pl.kernel(
    _body,
    out_type=jax.ShapeDtypeStruct((S, D), jnp.float32),
    mesh=_mesh,
    scratch_types=[
        pltpu.VMEM((544,), jnp.int32),
        pltpu.VMEM((CHUNK + 4, D), jnp.float32),
        pltpu.VMEM((CHUNK + 4, D), jnp.float32),
        pltpu.VMEM((SEG_PER_W, D), jnp.float32),
        pltpu.SemaphoreType.DMA,
        pltpu.SemaphoreType.DMA,
    ],
)


def kernel(x, batch, W, b):
    batch32 = batch.astype(jnp.int32)
    ids = jnp.arange(S + 1, dtype=jnp.int32)
    starts = jnp.searchsorted(batch32, ids).astype(jnp.int32)
    starts = jnp.concatenate([starts, jnp.zeros((31,), jnp.int32)])
    wpad = jnp.zeros((128, D), jnp.float32).at[0].set(
        W.reshape(-1).astype(jnp.float32))
    out2 = _sc_call(x, starts)
    out1 = _tc_call(x, batch32.reshape(NB, 1, BR), wpad,
                    b.astype(jnp.float32).reshape(1, 1))
    return jnp.concatenate([out1, out2], axis=1)
